# baseline (speedup 1.0000x reference)
import functools

import jax
import jax.numpy as jnp
from jax import lax
from jax.experimental import pallas as pl
from jax.experimental.pallas import tpu as pltpu
from jax.experimental.pallas import tpu_sc as plsc

BATCH = 16384
DIM = 32
NC = 2
NS = 16
NW = NC * NS
BPW = BATCH // NW
RING = 8


def _phase(sidx_hbm, perm_hbm, tblA, tblB, outAB,
           sv, pv, stgA, stgB, sog, ucols, sems, wid):
    row0 = wid * (BPW // 128)
    pltpu.sync_copy(sidx_hbm.at[pl.ds(row0, BPW // 128)], sv)
    pltpu.sync_copy(perm_hbm.at[pl.ds(row0, BPW // 128)], pv)

    bcarry = (jnp.int32(-1), jnp.int32(0))
    for j in range(BPW // 128):
        def build(g, carry):
            prev, n = carry
            idx16 = sv[j, pl.ds(g * 16, 16)]
            for l in range(16):
                col = idx16[l] >> 7
                is_new = col != prev

                @pl.when(is_new)
                def _():
                    ucols[n] = col

                n = jnp.where(is_new, n + 1, n)
                prev = col
            return prev, n

        bcarry = lax.fori_loop(0, 8, build, bcarry)
    n_cols = bcarry[1]

    def fire(k, slot):
        col = pl.multiple_of(ucols[k] * 128, 128)
        pltpu.async_copy(tblA.at[:, pl.ds(col, 128)],
                         stgA.at[pl.ds(slot * DIM, DIM), :], sems.at[slot])
        pltpu.async_copy(tblB.at[:, pl.ds(col, 128)],
                         stgB.at[pl.ds(slot * DIM, DIM), :], sems.at[slot])

    def prime(k, carry):
        fire(k, k)
        return carry

    lax.fori_loop(0, jnp.minimum(n_cols, RING), prime, 0)

    iota16 = lax.iota(jnp.int32, 16)

    carry = (jnp.int32(-1), jnp.int32(-1))
    for j in range(BPW // 128):
        def walk(g2, carry):
            prev, ptr = carry
            idx16 = sv[j, pl.ds(g2 * 16, 16)]
            for l in range(16):
                idx = idx16[l]
                col = idx >> 7
                is_new = col != prev
                ptr = jnp.where(is_new, ptr + 1, ptr)

                @pl.when(is_new)
                def _():
                    nxt = ptr + RING - 1

                    @pl.when((ptr >= 1) & (nxt < n_cols))
                    def _():
                        fire(nxt, nxt % RING)

                    slot_w = ptr % RING
                    pltpu.make_async_copy(
                        tblA.at[:, pl.ds(0, 128)],
                        stgA.at[pl.ds(slot_w * DIM, DIM), :],
                        sems.at[slot_w]).wait()
                    pltpu.make_async_copy(
                        tblB.at[:, pl.ds(0, 128)],
                        stgB.at[pl.ds(slot_w * DIM, DIM), :],
                        sems.at[slot_w]).wait()

                slot = ptr % RING
                lane = idx & 127
                lanes = jnp.full((16,), lane, jnp.int32)
                e = g2 * 16 + l
                for h in range(2):
                    rows = slot * DIM + h * 16 + iota16
                    sog[e, pl.ds(h * 16, 16)] = plsc.load_gather(
                        stgA, [rows, lanes])
                    sog[e, pl.ds(DIM + h * 16, 16)] = plsc.load_gather(
                        stgB, [rows, lanes])
                prev = col
            return prev, ptr

        carry = lax.fori_loop(0, 8, walk, carry)
        pltpu.sync_copy(sog, outAB.at[pv.at[j]])


def _sc_gather_body(su_hbm, pu_hbm, si_hbm, pi_hbm,
                    eugT, eumT, eigT, eimT,
                    out_u, out_i,
                    sv, pv, stgA, stgB, sog, ucols, sems):
    wid = lax.axis_index("s") * NC + lax.axis_index("c")
    _phase(su_hbm, pu_hbm, eugT, eumT, out_u,
           sv, pv, stgA, stgB, sog, ucols, sems, wid)
    _phase(si_hbm, pi_hbm, eigT, eimT, out_i,
           sv, pv, stgA, stgB, sog, ucols, sems, wid)


@jax.jit
def _sc_gather(su2d, pu2d, si2d, pi2d, eugT, eumT, eigT, eimT):
    mesh = plsc.VectorSubcoreMesh(core_axis_name="c", subcore_axis_name="s")
    out = jax.ShapeDtypeStruct((BATCH, 128), jnp.float32)
    fn = pl.kernel(
        _sc_gather_body,
        mesh=mesh,
        compiler_params=pltpu.CompilerParams(needs_layout_passes=False),
        out_type=[out, out],
        scratch_types=[
            pltpu.VMEM((BPW // 128, 128), jnp.int32),
            pltpu.VMEM((BPW // 128, 128), jnp.int32),
            pltpu.VMEM((RING * DIM, 128), jnp.float32),
            pltpu.VMEM((RING * DIM, 128), jnp.float32),
            pltpu.VMEM((128, 128), jnp.float32),
            pltpu.SMEM((BPW,), jnp.int32),
            pltpu.SemaphoreType.DMA((RING,)),
        ],
    )
    return fn(su2d, pu2d, si2d, pi2d, eugT, eumT, eigT, eimT)


def _tc_body(ou, oi, W1, b1, W2, b2, W3, b3, wpg, wpm, bp, out):
    ug = ou[:, :DIM]
    um = ou[:, DIM:2 * DIM]
    ig = oi[:, :DIM]
    im = oi[:, DIM:2 * DIM]
    x = jnp.concatenate([um, im], axis=1)
    h = jnp.maximum(jnp.dot(x, W1[...], preferred_element_type=jnp.float32) + b1[...], 0.0)
    h = jnp.maximum(jnp.dot(h, W2[...], preferred_element_type=jnp.float32) + b2[...], 0.0)
    h = jnp.maximum(jnp.dot(h, W3[...], preferred_element_type=jnp.float32) + b3[...], 0.0)
    pred = (jnp.sum(ug * ig * wpg[...], axis=1) + jnp.sum(h * wpm[...], axis=1)
            + bp[0, 0])
    out[...] = pred


def _tc_call(ou, oi, W1, b1, W2, b2, W3, b3, wpg, wpm, bp):
    nblk = 8
    blk = BATCH // nblk
    row_spec = pl.BlockSpec((blk, 128), lambda i: (i, 0))

    def whole(a):
        return pl.BlockSpec(a.shape, lambda i: (0,) * a.ndim)

    return pl.pallas_call(
        _tc_body,
        grid=(nblk,),
        in_specs=[row_spec, row_spec,
                  whole(W1), whole(b1), whole(W2), whole(b2),
                  whole(W3), whole(b3), whole(wpg), whole(wpm), whole(bp)],
        out_specs=pl.BlockSpec((blk,), lambda i: (i,)),
        out_shape=jax.ShapeDtypeStruct((BATCH,), jnp.float32),
    )(ou, oi, W1, b1, W2, b2, W3, b3, wpg, wpm, bp)


def kernel(user, item, eu_gmf, ei_gmf, eu_mlp, ei_mlp,
           W1, b1, W2, b2, W3, b3, Wp, bp):
    user = user.astype(jnp.int32)
    item = item.astype(jnp.int32)
    pos = lax.iota(jnp.int32, BATCH)
    keys = jnp.stack([user, item])
    vals = jnp.stack([pos, pos])
    sk, sp = lax.sort([keys, vals], dimension=1, num_keys=1)
    r2 = lambda a: a.reshape(BATCH // 128, 128)
    ou, oi = _sc_gather(r2(sk[0]), r2(sp[0]), r2(sk[1]), r2(sp[1]),
                        eu_gmf.T, eu_mlp.T, ei_gmf.T, ei_mlp.T)
    wpg = Wp[:DIM].reshape(1, DIM)
    wpm = Wp[DIM:].reshape(1, 16)
    return _tc_call(ou, oi,
                    W1, b1.reshape(1, -1), W2, b2.reshape(1, -1),
                    W3, b3.reshape(1, -1), wpg, wpm, bp.reshape(1, 1))


# speedup vs baseline: 1.2499x; 1.2499x over previous
import functools

import jax
import jax.numpy as jnp
from jax import lax
from jax.experimental import pallas as pl
from jax.experimental.pallas import tpu as pltpu
from jax.experimental.pallas import tpu_sc as plsc

BATCH = 16384
DIM = 32
NC = 2
NS = 16
NW = NC * NS
BPW = BATCH // NW
RING = 8


def _phase(sidx_hbm, perm_hbm, tblA, tblB, outAB,
           sv, pv, stgA, stgB, sog, ucols, sems, wid):
    row0 = wid * (BPW // 128)
    pltpu.sync_copy(sidx_hbm.at[pl.ds(row0, BPW // 128)], sv)
    pltpu.sync_copy(perm_hbm.at[pl.ds(row0, BPW // 128)], pv)

    bcarry = (jnp.int32(-1), jnp.int32(0))
    for j in range(BPW // 128):
        def build(g, carry):
            prev, n = carry
            idx16 = sv[j, pl.ds(g * 16, 16)]
            for l in range(16):
                col = idx16[l] >> 7
                is_new = col != prev

                @pl.when(is_new)
                def _():
                    ucols[n] = col

                n = jnp.where(is_new, n + 1, n)
                prev = col
            return prev, n

        bcarry = lax.fori_loop(0, 8, build, bcarry)
    n_cols = bcarry[1]

    def fire(k, slot):
        col = pl.multiple_of(ucols[k] * 128, 128)
        pltpu.async_copy(tblA.at[:, pl.ds(col, 128)],
                         stgA.at[pl.ds(slot * DIM, DIM), :], sems.at[slot])
        pltpu.async_copy(tblB.at[:, pl.ds(col, 128)],
                         stgB.at[pl.ds(slot * DIM, DIM), :], sems.at[slot])

    def prime(k, carry):
        fire(k, k)
        return carry

    lax.fori_loop(0, jnp.minimum(n_cols, RING), prime, 0)

    iota16 = lax.iota(jnp.int32, 16)

    carry = (jnp.int32(-1), jnp.int32(-1))
    for j in range(BPW // 128):
        def walk(g2, carry):
            prev, ptr = carry
            idx16 = sv[j, pl.ds(g2 * 16, 16)]
            for l in range(16):
                idx = idx16[l]
                col = idx >> 7
                is_new = col != prev
                ptr = jnp.where(is_new, ptr + 1, ptr)

                @pl.when(is_new)
                def _():
                    nxt = ptr + RING - 1

                    @pl.when((ptr >= 1) & (nxt < n_cols))
                    def _():
                        fire(nxt, nxt % RING)

                    slot_w = ptr % RING
                    pltpu.make_async_copy(
                        tblA.at[:, pl.ds(0, 128)],
                        stgA.at[pl.ds(slot_w * DIM, DIM), :],
                        sems.at[slot_w]).wait()
                    pltpu.make_async_copy(
                        tblB.at[:, pl.ds(0, 128)],
                        stgB.at[pl.ds(slot_w * DIM, DIM), :],
                        sems.at[slot_w]).wait()

                slot = ptr % RING
                lane = idx & 127
                lanes = jnp.full((16,), lane, jnp.int32)
                e = g2 * 16 + l
                for h in range(2):
                    rows = slot * DIM + h * 16 + iota16
                    sog[e, pl.ds(h * 16, 16)] = plsc.load_gather(
                        stgA, [rows, lanes])
                    sog[e, pl.ds(DIM + h * 16, 16)] = plsc.load_gather(
                        stgB, [rows, lanes])
                prev = col
            return prev, ptr

        carry = lax.fori_loop(0, 8, walk, carry)
        pltpu.sync_copy(sog, outAB.at[pv.at[j]])


def _sc_gather_body(su_hbm, pu_hbm, si_hbm, pi_hbm,
                    eugT, eumT, eigT, eimT,
                    out_u, out_i,
                    sv, pv, stgA, stgB, sog, ucols, sems):
    wid = lax.axis_index("s") * NC + lax.axis_index("c")
    _phase(su_hbm, pu_hbm, eugT, eumT, out_u,
           sv, pv, stgA, stgB, sog, ucols, sems, wid)
    _phase(si_hbm, pi_hbm, eigT, eimT, out_i,
           sv, pv, stgA, stgB, sog, ucols, sems, wid)


@jax.jit
def _sc_gather(su2d, pu2d, si2d, pi2d, eugT, eumT, eigT, eimT):
    mesh = plsc.VectorSubcoreMesh(core_axis_name="c", subcore_axis_name="s")
    out = jax.ShapeDtypeStruct((BATCH, 128), jnp.float32)
    fn = pl.kernel(
        _sc_gather_body,
        mesh=mesh,
        compiler_params=pltpu.CompilerParams(needs_layout_passes=False),
        out_type=[out, out],
        scratch_types=[
            pltpu.VMEM((BPW // 128, 128), jnp.int32),
            pltpu.VMEM((BPW // 128, 128), jnp.int32),
            pltpu.VMEM((RING * DIM, 128), jnp.float32),
            pltpu.VMEM((RING * DIM, 128), jnp.float32),
            pltpu.VMEM((128, 128), jnp.float32),
            pltpu.SMEM((BPW,), jnp.int32),
            pltpu.SemaphoreType.DMA((RING,)),
        ],
    )
    return fn(su2d, pu2d, si2d, pi2d, eugT, eumT, eigT, eimT)


def _tc_body(ou, oi, W1, b1, W2, b2, W3, b3, wpg, wpm, bp, out):
    ug = ou[:, :DIM]
    um = ou[:, DIM:2 * DIM]
    ig = oi[:, :DIM]
    im = oi[:, DIM:2 * DIM]
    x = jnp.concatenate([um, im], axis=1)
    h = jnp.maximum(jnp.dot(x, W1[...], preferred_element_type=jnp.float32) + b1[...], 0.0)
    h = jnp.maximum(jnp.dot(h, W2[...], preferred_element_type=jnp.float32) + b2[...], 0.0)
    h = jnp.maximum(jnp.dot(h, W3[...], preferred_element_type=jnp.float32) + b3[...], 0.0)
    pred = (jnp.sum(ug * ig * wpg[...], axis=1) + jnp.sum(h * wpm[...], axis=1)
            + bp[0, 0])
    out[...] = pred


def _tc_call(ou, oi, W1, b1, W2, b2, W3, b3, wpg, wpm, bp):
    nblk = 8
    blk = BATCH // nblk
    row_spec = pl.BlockSpec((blk, 128), lambda i: (i, 0))

    def whole(a):
        return pl.BlockSpec(a.shape, lambda i: (0,) * a.ndim)

    return pl.pallas_call(
        _tc_body,
        grid=(nblk,),
        in_specs=[row_spec, row_spec,
                  whole(W1), whole(b1), whole(W2), whole(b2),
                  whole(W3), whole(b3), whole(wpg), whole(wpm), whole(bp)],
        out_specs=pl.BlockSpec((blk,), lambda i: (i,)),
        out_shape=jax.ShapeDtypeStruct((BATCH,), jnp.float32),
    )(ou, oi, W1, b1, W2, b2, W3, b3, wpg, wpm, bp)


def kernel(user, item, eu_gmf, ei_gmf, eu_mlp, ei_mlp,
           W1, b1, W2, b2, W3, b3, Wp, bp):
    user = user.astype(jnp.int32)
    item = item.astype(jnp.int32)
    pos = lax.iota(jnp.int32, BATCH)
    su, pu = lax.sort([user, pos], num_keys=1)
    si, pi = lax.sort([item, pos], num_keys=1)
    r2 = lambda a: a.reshape(BATCH // 128, 128)
    ou, oi = _sc_gather(r2(su), r2(pu), r2(si), r2(pi),
                        eu_gmf.T, eu_mlp.T, ei_gmf.T, ei_mlp.T)
    wpg = Wp[:DIM].reshape(1, DIM)
    wpm = Wp[DIM:].reshape(1, 16)
    return _tc_call(ou, oi,
                    W1, b1.reshape(1, -1), W2, b2.reshape(1, -1),
                    W3, b3.reshape(1, -1), wpg, wpm, bp.reshape(1, 1))
